# baseline (speedup 1.0000x reference)
import jax
import jax.numpy as jnp
from jax import lax
from jax.experimental import pallas as pl
from jax.experimental.pallas import tpu as pltpu
from jax.experimental.pallas import tpu_sc as plsc

_B, _C, _H, _W = 8, 192, 224, 224
_N = _B * _C * _H
_G = _N // 8
_NC, _NS = 2, 16
_NW = _NC * _NS
_RPW = _N // _NW
_GPW = _G // _NW
_L = 16
_CG = 56
_NCHUNK = _GPW // _CG
_CROWS = _CG * 8


def _sc_body(x_hbm, w_hbm, out_hbm, w_v, buf0, buf1, acc_v, sem0, sem1):
    wid = lax.axis_index("s") * _NC + lax.axis_index("c")
    gbase = wid * _GPW
    nbase = wid * _RPW

    pltpu.sync_copy(w_hbm, w_v)
    lane = lax.iota(jnp.int32, _L)
    best_v = w_v[pl.ds(0, _L)]
    best_i = lane
    for i in range(1, _W // _L):
        vals = w_v[pl.ds(i * _L, _L)]
        upd = vals > best_v
        best_v = jnp.where(upd, vals, best_v)
        best_i = jnp.where(upd, lane + i * _L, best_i)
    m = jnp.max(best_v)
    idx = jnp.min(jnp.where(best_v == m, best_i, jnp.int32(2**30)))

    t128 = pl.multiple_of((idx // 128) * 128, 128)
    l = idx % 128
    l_vec = jnp.broadcast_to(l, (_L,))

    bufs = (buf0, buf1)
    sems = (sem0, sem1)

    def chunk_copy(c):
        return pltpu.make_async_copy(
            x_hbm.at[pl.ds((gbase + c * _CG) * 8, _CROWS), pl.ds(t128, 128)],
            bufs[c % 2],
            sems[c % 2],
        )

    chunk_copy(0).start()
    for c in range(_NCHUNK):
        if c + 1 < _NCHUNK:
            chunk_copy(c + 1).start()
        chunk_copy(c).wait()
        buf = bufs[c % 2]
        for k in range(_CROWS // _L):
            vals = plsc.load_gather(buf, [lane + k * _L, l_vec])
            acc_v[pl.ds(c * _CROWS + k * _L, _L)] = vals

    pltpu.sync_copy(acc_v, out_hbm.at[pl.ds(nbase, _RPW)])


def kernel(x, w):
    x2 = x.reshape(_N, _W)
    wf = w.reshape(_W)
    mesh = plsc.VectorSubcoreMesh(core_axis_name="c", subcore_axis_name="s",
                                  num_cores=_NC, num_subcores=_NS)
    out = pl.kernel(
        _sc_body,
        out_type=jax.ShapeDtypeStruct((_N,), jnp.float32),
        mesh=mesh,
        scratch_types=[
            pltpu.VMEM((_W,), jnp.float32),
            pltpu.VMEM((_CROWS, 128), jnp.float32),
            pltpu.VMEM((_CROWS, 128), jnp.float32),
            pltpu.VMEM((_RPW,), jnp.float32),
            pltpu.SemaphoreType.DMA,
            pltpu.SemaphoreType.DMA,
        ],
        compiler_params=pltpu.CompilerParams(use_tc_tiling_on_sc=True,
                                             needs_layout_passes=False),
    )(x2, wf)
    return out.reshape(_B, _C, _H)[..., None]


# speedup vs baseline: 1.0263x; 1.0263x over previous
import jax
import jax.numpy as jnp
from jax import lax
from jax.experimental import pallas as pl
from jax.experimental.pallas import tpu as pltpu
from jax.experimental.pallas import tpu_sc as plsc

_B, _C, _H, _W = 8, 192, 224, 224
_N = _B * _C * _H
_BC = _B * _C
_NC, _NS = 2, 16
_NW = _NC * _NS
_RPW = _N // _NW
_BCW = _BC // _NW
_L = 16
_CG = 56
_CROWS = _CG * 8
_NCHUNK = _RPW // _CROWS


def _sc_body(x_hbm, w_hbm, out_hbm, w_v, buf0, buf1, acc_v, sem0, sem1):
    wid = lax.axis_index("s") * _NC + lax.axis_index("c")
    nbase = wid * _RPW

    pltpu.sync_copy(w_hbm, w_v)
    lane = lax.iota(jnp.int32, _L)
    best_v = w_v[pl.ds(0, _L)]
    best_i = lane
    for i in range(1, _W // _L):
        vals = w_v[pl.ds(i * _L, _L)]
        upd = vals > best_v
        best_v = jnp.where(upd, vals, best_v)
        best_i = jnp.where(upd, lane + i * _L, best_i)
    m = jnp.max(best_v)
    idx = jnp.min(jnp.where(best_v == m, best_i, jnp.int32(2**30)))

    t128 = pl.multiple_of((idx // 128) * 128, 128)
    l = idx % 128
    l_vec = jnp.broadcast_to(l, (_L,))

    bufs = (buf0, buf1)
    sems = (sem0, sem1)

    def chunk_copy(c):
        return pltpu.make_async_copy(
            x_hbm.at[pl.ds(nbase + c * _CROWS, _CROWS), pl.ds(t128, 128)],
            bufs[c % 2],
            sems[c % 2],
        )

    chunk_copy(0).start()
    for c in range(_NCHUNK):
        if c + 1 < _NCHUNK:
            chunk_copy(c + 1).start()
        chunk_copy(c).wait()
        buf = bufs[c % 2]
        for k in range(_CROWS // _L):
            vals = plsc.load_gather(buf, [lane + k * _L, l_vec])
            off = c * _CROWS + k * _L
            acc_v[off // _H, pl.ds(off % _H, _L)] = vals

    pltpu.sync_copy(acc_v, out_hbm.at[pl.ds(wid * _BCW, _BCW)])


def kernel(x, w):
    x2 = x.reshape(_N, _W)
    wf = w.reshape(_W)
    mesh = plsc.VectorSubcoreMesh(core_axis_name="c", subcore_axis_name="s",
                                  num_cores=_NC, num_subcores=_NS)
    out = pl.kernel(
        _sc_body,
        out_type=jax.ShapeDtypeStruct((_BC, _H), jnp.float32),
        mesh=mesh,
        scratch_types=[
            pltpu.VMEM((_W,), jnp.float32),
            pltpu.VMEM((_CROWS, 128), jnp.float32),
            pltpu.VMEM((_CROWS, 128), jnp.float32),
            pltpu.VMEM((_BCW, _H), jnp.float32),
            pltpu.SemaphoreType.DMA,
            pltpu.SemaphoreType.DMA,
        ],
        compiler_params=pltpu.CompilerParams(use_tc_tiling_on_sc=True,
                                             needs_layout_passes=False),
    )(x2, wf)
    return out.reshape(_B, _C, _H)[..., None]
